# baseline (speedup 1.0000x reference)
import functools

import jax
import jax.numpy as jnp
from jax import lax
from jax.experimental import pallas as pl
from jax.experimental.pallas import tpu as pltpu
from jax.experimental.pallas import tpu_sc as plsc

GRID = 4
WIN = 3
STRIDE = GRID - WIN + 1
CT_SPLIT = 48
LANES = 16


def _band_sum_kernel(x_ref, o_ref):
    o_ref[0, 0, 0, :] = jnp.sum(x_ref[...], axis=(0, 1, 2))


def _select_kernel(tc_ref, sc_ref, o_ref):
    x = tc_ref[...] + sc_ref[...]
    B, G, W = x.shape
    gw = W // GRID
    lane = jax.lax.broadcasted_iota(jnp.int32, (B, W), 1)
    ws = []
    for i in range(STRIDE):
        rows = jnp.sum(x[:, i:i + WIN, :], axis=1)
        for j in range(STRIDE):
            m = (lane >= j * gw) & (lane < (j + WIN) * gw)
            ws.append(jnp.sum(jnp.where(m, rows, 0.0), axis=1))
    best_val = ws[0]
    best_idx = jnp.zeros((B,), jnp.int32)
    for k in range(1, STRIDE * STRIDE):
        better = ws[k] > best_val
        best_val = jnp.where(better, ws[k], best_val)
        best_idx = jnp.where(better, k, best_idx)
    coords = jnp.concatenate(
        [(best_idx // STRIDE)[:, None], (best_idx % STRIDE)[:, None]], axis=1)
    o_ref[...] = coords.astype(jnp.int32)


def _make_sc_band_sum(B, C, H, W):
    band = H // GRID
    chunk = band * W
    n_acc = W // LANES
    cs = C - CT_SPLIT
    mesh = plsc.VectorSubcoreMesh(core_axis_name="c", subcore_axis_name="s")

    @functools.partial(
        pl.kernel,
        mesh=mesh,
        out_type=jax.ShapeDtypeStruct((B * GRID, W), jnp.float32),
        scratch_types=[
            pltpu.VMEM((band, W), jnp.float32),
            pltpu.VMEM((band, W), jnp.float32),
            pltpu.VMEM((W,), jnp.float32),
            pltpu.SemaphoreType.DMA,
            pltpu.SemaphoreType.DMA,
        ],
    )
    def sc_band_sum(x_hbm, out_hbm, buf0, buf1, acc_vmem, sem0, sem1):
        wid = lax.axis_index("c") * 16 + lax.axis_index("s")
        b = wid // GRID
        gi = wid % GRID
        h0 = gi * band

        bufs = (buf0, buf1)
        sems = (sem0, sem1)
        copies = [None, None]
        copies[0] = pltpu.make_async_copy(
            x_hbm.at[b, CT_SPLIT, pl.ds(h0, band)], bufs[0], sems[0])
        copies[0].start()

        accs = tuple(jnp.zeros((LANES,), jnp.float32) for _ in range(n_acc))
        for ci in range(cs):
            par = ci % 2
            if ci + 1 < cs:
                copies[(ci + 1) % 2] = pltpu.make_async_copy(
                    x_hbm.at[b, CT_SPLIT + ci + 1, pl.ds(h0, band)],
                    bufs[(ci + 1) % 2], sems[(ci + 1) % 2])
                copies[(ci + 1) % 2].start()
            copies[par].wait()

            def rows_body(r, accs, cur=bufs[par]):
                return tuple(
                    accs[j] + cur[r, pl.ds(j * LANES, LANES)]
                    for j in range(n_acc))

            accs = lax.fori_loop(0, band, rows_body, accs)

        for j in range(n_acc):
            acc_vmem[pl.ds(j * LANES, LANES)] = accs[j]
        pltpu.sync_copy(acc_vmem, out_hbm.at[wid])

    return sc_band_sum


def kernel(sampling_map):
    B, C, H, W = sampling_map.shape
    band = H // GRID

    sc_partial = _make_sc_band_sum(B, C, H, W)(sampling_map)
    sc_partial = sc_partial.reshape(B, GRID, W)

    tc_partial = pl.pallas_call(
        _band_sum_kernel,
        grid=(B, GRID),
        in_specs=[pl.BlockSpec((1, CT_SPLIT, band, W), lambda b, g: (b, 0, g, 0))],
        out_specs=pl.BlockSpec((1, 1, 1, W), lambda b, g: (b, g, 0, 0)),
        out_shape=jax.ShapeDtypeStruct((B, GRID, 1, W), jnp.float32),
    )(sampling_map)
    tc_partial = tc_partial.reshape(B, GRID, W)

    coords = pl.pallas_call(
        _select_kernel,
        in_specs=[
            pl.BlockSpec((B, GRID, W), lambda: (0, 0, 0)),
            pl.BlockSpec((B, GRID, W), lambda: (0, 0, 0)),
        ],
        out_specs=pl.BlockSpec((B, 2), lambda: (0, 0)),
        out_shape=jax.ShapeDtypeStruct((B, 2), jnp.int32),
    )(tc_partial, sc_partial)

    return coords.reshape(B, 1, 2)


# speedup vs baseline: 1.2130x; 1.2130x over previous
import jax
import jax.numpy as jnp
from jax.experimental import pallas as pl

GRID = 4
WIN = 3
STRIDE = GRID - WIN + 1
NC = 2


def _band_sum_kernel(x_ref, o_ref):
    s = jnp.sum(x_ref[...], axis=(0, 1, 2))

    @pl.when(pl.program_id(2) == 0)
    def _init():
        o_ref[0, 0, 0, :] = s

    @pl.when(pl.program_id(2) != 0)
    def _acc():
        o_ref[0, 0, 0, :] += s


def _select_kernel(r_ref, o_ref):
    x = r_ref[...]
    B, G, W = x.shape
    gw = W // GRID
    lane = jax.lax.broadcasted_iota(jnp.int32, (B, W), 1)
    ws = []
    for i in range(STRIDE):
        rows = jnp.sum(x[:, i:i + WIN, :], axis=1)
        for j in range(STRIDE):
            m = (lane >= j * gw) & (lane < (j + WIN) * gw)
            ws.append(jnp.sum(jnp.where(m, rows, 0.0), axis=1))
    best_val = ws[0]
    best_idx = jnp.zeros((B,), jnp.int32)
    for k in range(1, STRIDE * STRIDE):
        better = ws[k] > best_val
        best_val = jnp.where(better, ws[k], best_val)
        best_idx = jnp.where(better, k, best_idx)
    coords = jnp.concatenate(
        [(best_idx // STRIDE)[:, None], (best_idx % STRIDE)[:, None]], axis=1)
    o_ref[...] = coords.astype(jnp.int32)


def kernel(sampling_map):
    B, C, H, W = sampling_map.shape
    band = H // GRID
    cchunk = C // NC

    band_sums = pl.pallas_call(
        _band_sum_kernel,
        grid=(B, GRID, NC),
        in_specs=[pl.BlockSpec((1, cchunk, band, W),
                               lambda b, g, c: (b, c, g, 0))],
        out_specs=pl.BlockSpec((1, 1, 1, W), lambda b, g, c: (b, g, 0, 0)),
        out_shape=jax.ShapeDtypeStruct((B, GRID, 1, W), jnp.float32),
    )(sampling_map)
    band_sums = band_sums.reshape(B, GRID, W)

    coords = pl.pallas_call(
        _select_kernel,
        in_specs=[pl.BlockSpec((B, GRID, W), lambda: (0, 0, 0))],
        out_specs=pl.BlockSpec((B, 2), lambda: (0, 0)),
        out_shape=jax.ShapeDtypeStruct((B, 2), jnp.int32),
    )(band_sums)

    return coords.reshape(B, 1, 2)
